# baseline (speedup 1.0000x reference)
import jax
import jax.numpy as jnp
from jax import lax
from jax.experimental import pallas as pl
from jax.experimental.pallas import tpu as pltpu
from jax.experimental.pallas import tpu_sc as plsc

_B = 1024
_C = 100000
_B_TC = 512
_BR = 64

_NC = 2
_NS = 16
_NW = _NC * _NS
_SC_ROWS = _B - _B_TC
_RPW = _SC_ROWS // _NW
_NT = 780
_TAIL0 = _NT * 128
_TAIL_W = _C - _TAIL0
_NBUF = 8


def _tc_body(y_ref, x_ref, o_ref):
    x = x_ref[...]
    yv = y_ref[...]
    c = x.shape[1]
    cols = jax.lax.broadcasted_iota(jnp.int32, x.shape, 1)
    oy = jnp.sum(jnp.where(cols == yv, x, 0.0), axis=1, keepdims=True)
    s = jnp.sum(jnp.maximum(x - (oy - 1.0), 0.0), axis=1, keepdims=True)
    o_ref[...] = (s - 1.0) / c


def _gather_body(y_sref, *refs):
    parts = []
    lanes = jax.lax.broadcasted_iota(jnp.int32, (1, 128), 1)
    i = pl.program_id(0)
    for k in range(8):
        yv = y_sref[_B_TC + i * 8 + k]
        xb = refs[k][k:k + 1, :]
        parts.append(
            jnp.sum(jnp.where(lanes == (yv & 127), xb, 0.0), axis=1,
                    keepdims=True))
    refs[8][...] = jnp.concatenate(parts, axis=0)


def _tail_body(oy_ref, sp_ref, x_ref, o_ref):
    x = x_ref[...]
    t = oy_ref[...] - 1.0
    cols = jax.lax.broadcasted_iota(jnp.int32, x.shape, 1)
    h = jnp.where(cols < _TAIL_W, jnp.maximum(x - t, 0.0), 0.0)
    part = jnp.sum(h, axis=1, keepdims=True)
    o_ref[...] = (sp_ref[...] + part - 1.0) / _C


def _sc_body(x_hbm, oy_hbm, o_hbm, oy_v, buf3, res_v, shared, sem):
    cid = lax.axis_index("c")
    sid = lax.axis_index("s")
    wid = cid * _NS + sid
    base = pl.multiple_of(_B_TC + wid * _RPW, 8)
    pltpu.sync_copy(oy_hbm, oy_v)
    lane = lax.iota(jnp.int32, 16)
    oy16 = oy_v[pl.ds(wid * _RPW, _RPW)]

    res = jnp.zeros((16,), jnp.float32)
    for g in range(_RPW // 8):
        rbase = pl.multiple_of(base + g * 8, 8)
        ts = [jnp.sum(jnp.where(lane == g * 8 + rr, oy16, 0.0)) - 1.0
              for rr in range(8)]

        def tile_copy(kt, slot):
            c0 = pl.multiple_of(kt * 128, 128)
            return pltpu.make_async_copy(
                x_hbm.at[pl.ds(rbase, 8), pl.ds(c0, 128)],
                buf3.at[slot], sem.at[slot])

        for s in range(_NBUF):
            tile_copy(s, s).start()

        def batch(gb, accs):
            new = accs
            for s in range(_NBUF):
                kt = gb * _NBUF + s
                tile_copy(kt, s).wait()

                def inner(j, a, s=s):
                    return tuple(
                        a[rr] + jnp.maximum(
                            buf3[s, rr, pl.ds(j * 16, 16)] - ts[rr], 0.0)
                        for rr in range(8))

                new = lax.fori_loop(0, 8, inner, new)

                @pl.when(kt + _NBUF < _NT)
                def _():
                    tile_copy(kt + _NBUF, s).start()
            return new

        accs = tuple(jnp.zeros((16,), jnp.float32) for _ in range(8))
        accs = lax.fori_loop(0, _NT // _NBUF, batch, accs)
        nfull = (_NT // _NBUF) * _NBUF
        for s in range(_NT - nfull):
            tile_copy(nfull + s, s).wait()

            def inner_t(j, a, s=s):
                return tuple(
                    a[rr] + jnp.maximum(
                        buf3[s, rr, pl.ds(j * 16, 16)] - ts[rr], 0.0)
                    for rr in range(8))

            accs = lax.fori_loop(0, 8, inner_t, accs)

        for rr in range(8):
            s_r = jnp.sum(accs[rr])
            res = jnp.where(lane == g * 8 + rr, s_r, res)
    res_v[...] = res
    pltpu.sync_copy(res_v, shared.at[pl.ds(sid * _RPW, _RPW)])
    plsc.subcore_barrier()

    @pl.when(sid == 0)
    def _():
        half = _SC_ROWS // _NC
        off = pl.multiple_of(cid * half, 8)
        pltpu.sync_copy(shared, o_hbm.at[pl.ds(off, half)])


def _sc_partial(output, oy):
    mesh = plsc.VectorSubcoreMesh(core_axis_name="c", subcore_axis_name="s")
    f = pl.kernel(
        _sc_body,
        out_type=jax.ShapeDtypeStruct((_SC_ROWS,), jnp.float32),
        mesh=mesh,
        scratch_types=[
            pltpu.VMEM((_SC_ROWS,), jnp.float32),
            pltpu.VMEM((_NBUF, 8, 128), jnp.float32),
            pltpu.VMEM((16,), jnp.float32),
            pltpu.VMEM_SHARED((_SC_ROWS // _NC,), jnp.float32),
            pltpu.SemaphoreType.DMA((_NBUF,)),
        ],
        compiler_params=pltpu.CompilerParams(needs_layout_passes=False),
    )
    return f(output, oy)


def kernel(output, y):
    b, c = output.shape
    y1 = y.astype(jnp.int32)
    y2 = y1.reshape(b, 1)

    loss_tc = pl.pallas_call(
        _tc_body,
        grid=(_B_TC // _BR,),
        in_specs=[
            pl.BlockSpec((_BR, 1), lambda i: (i, 0)),
            pl.BlockSpec((_BR, c), lambda i: (i, 0)),
        ],
        out_specs=pl.BlockSpec((_BR, 1), lambda i: (i, 0)),
        out_shape=jax.ShapeDtypeStruct((_B_TC, 1), jnp.float32),
    )(y2, output)

    in_specs = [
        pl.BlockSpec(
            (8, 128),
            (lambda i, y_sref, k=k: (
                _B_TC // 8 + i,
                jax.lax.shift_right_logical(y_sref[_B_TC + i * 8 + k], 7))))
        for k in range(8)
    ]
    oy = pl.pallas_call(
        _gather_body,
        grid_spec=pltpu.PrefetchScalarGridSpec(
            num_scalar_prefetch=1,
            grid=(_SC_ROWS // 8,),
            in_specs=in_specs,
            out_specs=pl.BlockSpec((8, 1), lambda i, y_sref: (i, 0)),
        ),
        out_shape=jax.ShapeDtypeStruct((_SC_ROWS, 1), jnp.float32),
    )(y1, *([output] * 8))

    sp = _sc_partial(output, oy.reshape(_SC_ROWS))

    loss_tail = pl.pallas_call(
        _tail_body,
        grid=(_SC_ROWS // 64,),
        in_specs=[
            pl.BlockSpec((64, 1), lambda i: (i, 0)),
            pl.BlockSpec((64, 1), lambda i: (i, 0)),
            pl.BlockSpec((64, 256),
                         lambda i: (_B_TC // 64 + i, _TAIL0 // 256)),
        ],
        out_specs=pl.BlockSpec((64, 1), lambda i: (i, 0)),
        out_shape=jax.ShapeDtypeStruct((_SC_ROWS, 1), jnp.float32),
    )(oy, sp.reshape(_SC_ROWS, 1), output)

    return jnp.concatenate([loss_tc.reshape(_B_TC),
                            loss_tail.reshape(_SC_ROWS)])


# speedup vs baseline: 1.2217x; 1.2217x over previous
import jax
import jax.numpy as jnp
from jax.experimental import pallas as pl
from jax.experimental.pallas import tpu as pltpu

_BR = 64


def _hinge_body(y_ref, x_ref, o_ref):
    x = x_ref[...]
    yv = y_ref[...]
    c = x.shape[1]
    cols = jax.lax.broadcasted_iota(jnp.int32, x.shape, 1)
    oy = jnp.sum(jnp.where(cols == yv, x, 0.0), axis=1, keepdims=True)
    s = jnp.sum(jnp.maximum(x - (oy - 1.0), 0.0), axis=1, keepdims=True)
    o_ref[...] = (s - 1.0) / c


def kernel(output, y):
    b, c = output.shape
    y2 = y.astype(jnp.int32).reshape(b, 1)
    out = pl.pallas_call(
        _hinge_body,
        grid=(b // _BR,),
        in_specs=[
            pl.BlockSpec((_BR, 1), lambda i: (i, 0)),
            pl.BlockSpec((_BR, c), lambda i: (i, 0)),
        ],
        out_specs=pl.BlockSpec((_BR, 1), lambda i: (i, 0)),
        out_shape=jax.ShapeDtypeStruct((b, 1), jnp.float32),
    )(y2, output)
    return out.reshape(b)
